# baseline (speedup 1.0000x reference)
import jax
import jax.numpy as jnp
import numpy as np
from jax.experimental import pallas as pl
from jax.experimental.pallas import tpu as pltpu

_ZONE_COLS = [
    np.array([1, 3, 4, 7, 8, 9, 10, 11, 16, 17, 18, 19, 20, 21]) - 1,
    np.array([17, 18, 19, 20, 21, 27, 28, 29, 30, 31, 36, 37, 38, 39, 40, 41]) - 1,
    np.array([37, 38, 39, 40, 41, 47, 48, 49, 50, 51]) - 1,
    np.array([56, 57, 58, 59, 62, 63]) - 1,
    np.array([59, 60, 61, 63, 64]) - 1,
    np.array([41, 42, 43, 44, 45, 46, 51, 52, 53, 54, 55, 56]) - 1,
    np.array([21, 22, 23, 24, 25, 31, 32, 33, 34, 35, 41, 42, 43, 44, 45, 46]) - 1,
    np.array([2, 5, 6, 11, 12, 13, 14, 15, 21, 22, 23, 24, 25, 26]) - 1,
]
_WIDTHS = [len(z) for z in _ZONE_COLS]

_N_ROWS = 16384
_N_COLS = 64
_CHUNK = 2048
_N_CHUNKS = _N_ROWS // _CHUNK


def _runs(cols):
    out = []
    start = int(cols[0])
    length = 1
    for a, b in zip(cols[:-1], cols[1:]):
        if int(b) == int(a) + 1:
            length += 1
        else:
            out.append((start, length))
            start = int(b)
            length = 1
    out.append((start, length))
    return out


_RUNS = [_runs(z) for z in _ZONE_COLS]


def _split_body(in_hbm, *rest):
    outs = rest[:8]
    in_bufs = rest[8]
    zone_bufs = rest[9:17]
    sem_in = rest[17]
    sem_out = rest[18]

    def in_copy(i, p):
        return pltpu.make_async_copy(
            in_hbm.at[pl.ds(i * _CHUNK, _CHUNK), :],
            in_bufs.at[p],
            sem_in.at[p],
        )

    def out_copy(i, p, z):
        return pltpu.make_async_copy(
            zone_bufs[z].at[p],
            outs[z].at[:, pl.ds(i * _CHUNK, _CHUNK)],
            sem_out.at[p, z],
        )

    in_copy(0, 0).start()
    for i in range(_N_CHUNKS):
        p = i % 2
        if i + 1 < _N_CHUNKS:
            in_copy(i + 1, (i + 1) % 2).start()
        in_copy(i, p).wait()
        xt = in_bufs[p].T
        if i >= 2:
            for z in range(8):
                out_copy(i - 2, p, z).wait()
        for z, runs in enumerate(_RUNS):
            zone_bufs[z][p] = jnp.concatenate(
                [xt[a : a + l, :] for (a, l) in runs], axis=0
            )
        for z in range(8):
            out_copy(i, p, z).start()
    for i in (_N_CHUNKS - 2, _N_CHUNKS - 1):
        for z in range(8):
            out_copy(i, i % 2, z).wait()


@jax.jit
def kernel(inputs):
    outs_t = pl.pallas_call(
        _split_body,
        in_specs=[pl.BlockSpec(memory_space=pl.ANY)],
        out_specs=tuple(
            pl.BlockSpec(memory_space=pl.ANY) for _ in _WIDTHS
        ),
        out_shape=tuple(
            jax.ShapeDtypeStruct((w, _N_ROWS), jnp.float32) for w in _WIDTHS
        ),
        scratch_shapes=[pltpu.VMEM((2, _CHUNK, _N_COLS), jnp.float32)]
        + [pltpu.VMEM((2, w, _CHUNK), jnp.float32) for w in _WIDTHS]
        + [
            pltpu.SemaphoreType.DMA((2,)),
            pltpu.SemaphoreType.DMA((2, 8)),
        ],
    )(inputs)
    return tuple(o.T for o in outs_t)


# speedup vs baseline: 1.0975x; 1.0975x over previous
import jax
import jax.numpy as jnp
import numpy as np
from jax.experimental import pallas as pl
from jax.experimental.pallas import tpu as pltpu

_ZONE_COLS = [
    np.array([1, 3, 4, 7, 8, 9, 10, 11, 16, 17, 18, 19, 20, 21]) - 1,
    np.array([17, 18, 19, 20, 21, 27, 28, 29, 30, 31, 36, 37, 38, 39, 40, 41]) - 1,
    np.array([37, 38, 39, 40, 41, 47, 48, 49, 50, 51]) - 1,
    np.array([56, 57, 58, 59, 62, 63]) - 1,
    np.array([59, 60, 61, 63, 64]) - 1,
    np.array([41, 42, 43, 44, 45, 46, 51, 52, 53, 54, 55, 56]) - 1,
    np.array([21, 22, 23, 24, 25, 31, 32, 33, 34, 35, 41, 42, 43, 44, 45, 46]) - 1,
    np.array([2, 5, 6, 11, 12, 13, 14, 15, 21, 22, 23, 24, 25, 26]) - 1,
]
_WIDTHS = [len(z) for z in _ZONE_COLS]

_N_ROWS = 16384
_N_COLS = 64
_CHUNK = 4096
_N_CHUNKS = _N_ROWS // _CHUNK


def _runs(cols):
    out = []
    start = int(cols[0])
    length = 1
    for a, b in zip(cols[:-1], cols[1:]):
        if int(b) == int(a) + 1:
            length += 1
        else:
            out.append((start, length))
            start = int(b)
            length = 1
    out.append((start, length))
    return out


_RUNS = [_runs(z) for z in _ZONE_COLS]


def _split_body(in_hbm, *rest):
    outs = rest[:8]
    in_bufs = rest[8]
    zone_bufs = rest[9:17]
    sem_in = rest[17]
    sem_out = rest[18]

    def in_copy(i, p):
        return pltpu.make_async_copy(
            in_hbm.at[pl.ds(i * _CHUNK, _CHUNK), :],
            in_bufs.at[p],
            sem_in.at[p],
        )

    def out_copy(i, p, z):
        return pltpu.make_async_copy(
            zone_bufs[z].at[p],
            outs[z].at[:, pl.ds(i * _CHUNK, _CHUNK)],
            sem_out.at[p, z],
        )

    in_copy(0, 0).start()
    for i in range(_N_CHUNKS):
        p = i % 2
        if i + 1 < _N_CHUNKS:
            in_copy(i + 1, (i + 1) % 2).start()
        in_copy(i, p).wait()
        xt = in_bufs[p].T
        if i >= 2:
            for z in range(8):
                out_copy(i - 2, p, z).wait()
        for z, runs in enumerate(_RUNS):
            zone_bufs[z][p] = jnp.concatenate(
                [xt[a : a + l, :] for (a, l) in runs], axis=0
            )
        for z in range(8):
            out_copy(i, p, z).start()
    for i in (_N_CHUNKS - 2, _N_CHUNKS - 1):
        for z in range(8):
            out_copy(i, i % 2, z).wait()


@jax.jit
def kernel(inputs):
    outs_t = pl.pallas_call(
        _split_body,
        in_specs=[pl.BlockSpec(memory_space=pl.ANY)],
        out_specs=tuple(
            pl.BlockSpec(memory_space=pl.ANY) for _ in _WIDTHS
        ),
        out_shape=tuple(
            jax.ShapeDtypeStruct((w, _N_ROWS), jnp.float32) for w in _WIDTHS
        ),
        scratch_shapes=[pltpu.VMEM((2, _CHUNK, _N_COLS), jnp.float32)]
        + [pltpu.VMEM((2, w, _CHUNK), jnp.float32) for w in _WIDTHS]
        + [
            pltpu.SemaphoreType.DMA((2,)),
            pltpu.SemaphoreType.DMA((2, 8)),
        ],
    )(inputs)
    return tuple(o.T for o in outs_t)


# speedup vs baseline: 1.1899x; 1.0842x over previous
import jax
import jax.numpy as jnp
import numpy as np
from jax.experimental import pallas as pl
from jax.experimental.pallas import tpu as pltpu

_ZONE_COLS = [
    np.array([1, 3, 4, 7, 8, 9, 10, 11, 16, 17, 18, 19, 20, 21]) - 1,
    np.array([17, 18, 19, 20, 21, 27, 28, 29, 30, 31, 36, 37, 38, 39, 40, 41]) - 1,
    np.array([37, 38, 39, 40, 41, 47, 48, 49, 50, 51]) - 1,
    np.array([56, 57, 58, 59, 62, 63]) - 1,
    np.array([59, 60, 61, 63, 64]) - 1,
    np.array([41, 42, 43, 44, 45, 46, 51, 52, 53, 54, 55, 56]) - 1,
    np.array([21, 22, 23, 24, 25, 31, 32, 33, 34, 35, 41, 42, 43, 44, 45, 46]) - 1,
    np.array([2, 5, 6, 11, 12, 13, 14, 15, 21, 22, 23, 24, 25, 26]) - 1,
]
_WIDTHS = [len(z) for z in _ZONE_COLS]

_N_ROWS = 16384
_N_COLS = 64
_CHUNK = 8192
_N_CHUNKS = _N_ROWS // _CHUNK


def _runs(cols):
    out = []
    start = int(cols[0])
    length = 1
    for a, b in zip(cols[:-1], cols[1:]):
        if int(b) == int(a) + 1:
            length += 1
        else:
            out.append((start, length))
            start = int(b)
            length = 1
    out.append((start, length))
    return out


_RUNS = [_runs(z) for z in _ZONE_COLS]


def _split_body(in_hbm, *rest):
    outs = rest[:8]
    in_bufs = rest[8]
    zone_bufs = rest[9:17]
    sem_in = rest[17]
    sem_out = rest[18]

    def in_copy(i, p):
        return pltpu.make_async_copy(
            in_hbm.at[pl.ds(i * _CHUNK, _CHUNK), :],
            in_bufs.at[p],
            sem_in.at[p],
        )

    def out_copy(i, p, z):
        return pltpu.make_async_copy(
            zone_bufs[z].at[p],
            outs[z].at[:, pl.ds(i * _CHUNK, _CHUNK)],
            sem_out.at[p, z],
        )

    in_copy(0, 0).start()
    for i in range(_N_CHUNKS):
        p = i % 2
        if i + 1 < _N_CHUNKS:
            in_copy(i + 1, (i + 1) % 2).start()
        in_copy(i, p).wait()
        xt = in_bufs[p].T
        if i >= 2:
            for z in range(8):
                out_copy(i - 2, p, z).wait()
        for z, runs in enumerate(_RUNS):
            zone_bufs[z][p] = jnp.concatenate(
                [xt[a : a + l, :] for (a, l) in runs], axis=0
            )
        for z in range(8):
            out_copy(i, p, z).start()
    for i in (_N_CHUNKS - 2, _N_CHUNKS - 1):
        for z in range(8):
            out_copy(i, i % 2, z).wait()


@jax.jit
def kernel(inputs):
    outs_t = pl.pallas_call(
        _split_body,
        in_specs=[pl.BlockSpec(memory_space=pl.ANY)],
        out_specs=tuple(
            pl.BlockSpec(memory_space=pl.ANY) for _ in _WIDTHS
        ),
        out_shape=tuple(
            jax.ShapeDtypeStruct((w, _N_ROWS), jnp.float32) for w in _WIDTHS
        ),
        scratch_shapes=[pltpu.VMEM((2, _CHUNK, _N_COLS), jnp.float32)]
        + [pltpu.VMEM((2, w, _CHUNK), jnp.float32) for w in _WIDTHS]
        + [
            pltpu.SemaphoreType.DMA((2,)),
            pltpu.SemaphoreType.DMA((2, 8)),
        ],
    )(inputs)
    return tuple(o.T for o in outs_t)
